# Initial kernel scaffold; baseline (speedup 1.0000x reference)
import jax
import jax.numpy as jnp
from jax.experimental import pallas as pl


def kernel(sigma, radiance, dt, numsteps_in, bkg_color, inference_only):
    raise NotImplementedError("write your pallas kernel here")


# speedup vs baseline: 21.9348x; 21.9348x over previous
import functools

import numpy as np
import jax
import jax.numpy as jnp
from jax import lax
from jax.experimental import pallas as pl
from jax.experimental.pallas import tpu as pltpu
from jax.experimental.pallas import tpu_sc as plsc

WIN = 640
RB = 64
_UPPER = np.triu(np.ones((WIN, WIN), dtype=np.float32))


def _round16(n: int) -> int:
    return (n + 15) // 16 * 16


def _make_render_body(tp: int):
    def body(ns_ref, bkg_ref, u_ref, pk_ref, rgb_ref, mask_ref):
        ns = ns_ref[...]
        steps = ns[:, 0:1]
        off = ns[:, 1:2]
        ws = jnp.minimum(jnp.bitwise_and(off, -16), tp - WIN)
        ls = off - ws
        sig = pk_ref[0]
        dtv = pk_ref[1]
        iota = lax.broadcasted_iota(jnp.int32, (RB, WIN), 1)
        valid = (iota >= ls) & (iota < ls + steps)
        x = jnp.where(valid, jnp.maximum(sig, 0.0) * dtv, 0.0)
        s = jnp.dot(x, u_ref[...], preferred_element_type=jnp.float32,
                    precision=lax.Precision.HIGHEST)
        w = jnp.exp(x - s) - jnp.exp(-s)
        s_end = s[:, WIN - 1:WIN]
        xl = jnp.sum(jnp.where(iota == ls + 511, x, 0.0), axis=1, keepdims=True)
        t_bkg = jnp.exp(xl - s_end)
        mask_ref[...] = 1.0 - jnp.exp(-s_end)
        for c in range(3):
            acc = jnp.sum(w * pk_ref[2 + c], axis=1, keepdims=True)
            rgb_ref[:, c:c + 1] = acc + t_bkg * bkg_ref[0, c]

    return body


def _render(pk, ns, bkg, tp):
    n_rays = ns.shape[0]
    grid = (n_rays // RB,)
    return pl.pallas_call(
        _make_render_body(tp),
        grid=grid,
        in_specs=[
            pl.BlockSpec((RB, 2), lambda i: (i, 0)),
            pl.BlockSpec((1, 3), lambda i: (0, 0)),
            pl.BlockSpec((WIN, WIN), lambda i: (0, 0)),
            pl.BlockSpec((5, RB, WIN), lambda i: (0, i, 0)),
        ],
        out_specs=[
            pl.BlockSpec((RB, 3), lambda i: (i, 0)),
            pl.BlockSpec((RB, 1), lambda i: (i, 0)),
        ],
        out_shape=[
            jax.ShapeDtypeStruct((n_rays, 3), jnp.float32),
            jax.ShapeDtypeStruct((n_rays, 1), jnp.float32),
        ],
    )(ns, bkg, jnp.asarray(_UPPER), pk)


def _sc_pack(packed5, offs, n_rays, tp):
    info = plsc.get_sparse_core_info()
    nw = info.num_cores * info.num_subcores
    rpw = n_rays // nw

    @functools.partial(
        pl.kernel,
        out_type=jax.ShapeDtypeStruct((5, n_rays, WIN), jnp.float32),
        mesh=plsc.VectorSubcoreMesh(core_axis_name="c", subcore_axis_name="s"),
        scratch_types=[pltpu.VMEM((rpw,), jnp.int32)],
        compiler_params=pltpu.CompilerParams(use_tc_tiling_on_sc=False),
    )
    def pack(src_hbm, offs_hbm, out_hbm, offs_v):
        c = lax.axis_index("c")
        s = lax.axis_index("s")
        wid = s * info.num_cores + c
        base = wid * rpw
        pltpu.sync_copy(offs_hbm.at[pl.ds(base, rpw)], offs_v)

        def body(g, carry):
            offv = offs_v[pl.ds(g * 16, 16)]
            for j in range(16):
                off = offv[j]
                ws = pl.multiple_of(
                    jnp.minimum(jnp.bitwise_and(off, -16), tp - WIN), 16)
                pltpu.sync_copy(src_hbm.at[:, pl.ds(ws, WIN)],
                                out_hbm.at[:, base + g * 16 + j, :])
            return carry

        lax.fori_loop(0, rpw // 16, body, 0)

    return pack(packed5, offs)


def kernel(sigma, radiance, dt, numsteps_in, bkg_color, inference_only):
    del inference_only
    n_rays = numsteps_in.shape[0]
    total = sigma.shape[0]
    tp = _round16(total)
    packed5 = jnp.concatenate(
        [sigma[None, :], dt[None, :], radiance.T], axis=0)
    packed5 = jnp.pad(packed5, ((0, 0), (0, tp - total)))
    offs = numsteps_in[:, 1]
    pk = _sc_pack(packed5, offs, n_rays, tp)
    rgb, mask = _render(pk, numsteps_in, bkg_color, tp)
    return rgb, mask.reshape(n_rays)
